# Initial kernel scaffold; baseline (speedup 1.0000x reference)
import jax
import jax.numpy as jnp
from jax.experimental import pallas as pl


def kernel(x, edge_index, W1, b1, W2, b2):
    raise NotImplementedError("write your pallas kernel here")


# speedup vs baseline: 3.4295x; 3.4295x over previous
import jax
import jax.numpy as jnp
from jax import lax
from jax.experimental import pallas as pl
from jax.experimental.pallas import tpu as pltpu
from jax.experimental.pallas import tpu_sc as plsc

N = 10000
E = 320000
D = 128
NC, NS = 2, 16
NW = NC * NS
CHUNK = 128
K = 80
PH = 2
KP = K // PH
E_PAD = NW * K * CHUNK
DUMMY = N
P = 10112
RPT = P // NS
ROW_CH = (128, 128, 128, 128, 120)
BR = 400

_f32 = jnp.float32


def _mesh():
    return plsc.VectorSubcoreMesh(
        core_axis_name="c", subcore_axis_name="s",
        num_cores=NC, num_subcores=NS)


def _deg_body(idx_hbm, out_hbm, idxv, ones_v, zbuf, tab_s):
    c = lax.axis_index("c")
    s = lax.axis_index("s")
    wid = s * NC + c

    def _init(i, carry):
        for t in range(D // 16):
            ones_v[i, pl.ds(16 * t, 16)] = jnp.ones((16,), _f32)
            zbuf[i, pl.ds(16 * t, 16)] = jnp.zeros((16,), _f32)
        return carry
    lax.fori_loop(0, CHUNK, _init, 0)

    base = s * RPT
    off = 0
    for sz in ROW_CH:
        pltpu.sync_copy(zbuf.at[pl.ds(0, sz)], tab_s.at[pl.ds(base + off, sz)])
        off += sz
    plsc.subcore_barrier()

    def _hist(j, carry):
        pltpu.sync_copy(ones_v, tab_s.at[idxv.at[j]], add=True)
        return carry

    for ph in range(PH):
        pltpu.sync_copy(idx_hbm.at[wid, pl.ds(ph * KP, KP)], idxv)
        lax.fori_loop(0, KP, _hist, 0)
    plsc.subcore_barrier()

    off = 0
    for sz in ROW_CH:
        pltpu.sync_copy(tab_s.at[pl.ds(base + off, sz)], zbuf.at[pl.ds(0, sz)])
        pltpu.sync_copy(zbuf.at[pl.ds(0, sz)], out_hbm.at[c, pl.ds(base + off, sz)])
        off += sz


def _make_deg_call():
    return pl.kernel(
        _deg_body,
        out_type=jax.ShapeDtypeStruct((NC, P, D), _f32),
        mesh=_mesh(),
        scratch_types=[
            pltpu.VMEM((KP, CHUNK), jnp.int32),
            pltpu.VMEM((CHUNK, D), _f32),
            pltpu.VMEM((CHUNK, D), _f32),
            pltpu.VMEM_SHARED((P, D), _f32),
        ],
    )


def _agg_body(h_hbm, src_hbm, dst_hbm, out_hbm,
              srcv, dstv, rows0, rows1, agg_s, sem0, sem1):
    c = lax.axis_index("c")
    s = lax.axis_index("s")
    wid = s * NC + c

    def _zero(i, carry):
        for t in range(D // 16):
            rows0[i, pl.ds(16 * t, 16)] = jnp.zeros((16,), _f32)
        return carry
    lax.fori_loop(0, CHUNK, _zero, 0)

    base = s * RPT
    off = 0
    for sz in ROW_CH:
        pltpu.sync_copy(rows0.at[pl.ds(0, sz)], agg_s.at[pl.ds(base + off, sz)])
        off += sz
    plsc.subcore_barrier()

    def _step(g, carry):
        j0 = 2 * g
        j1 = j0 + 1
        pltpu.make_async_copy(h_hbm.at[srcv.at[j0]], rows0, sem0).wait()
        pltpu.sync_copy(rows0, agg_s.at[dstv.at[j0]], add=True)
        pltpu.async_copy(h_hbm.at[srcv.at[j0 + 2]], rows0, sem0)
        pltpu.make_async_copy(h_hbm.at[srcv.at[j1]], rows1, sem1).wait()
        pltpu.sync_copy(rows1, agg_s.at[dstv.at[j1]], add=True)
        pltpu.async_copy(h_hbm.at[srcv.at[j1 + 2]], rows1, sem1)
        return carry

    for ph in range(PH):
        pltpu.sync_copy(src_hbm.at[wid, pl.ds(ph * KP, KP)], srcv)
        pltpu.sync_copy(dst_hbm.at[wid, pl.ds(ph * KP, KP)], dstv)
        pltpu.async_copy(h_hbm.at[srcv.at[0]], rows0, sem0)
        pltpu.async_copy(h_hbm.at[srcv.at[1]], rows1, sem1)
        lax.fori_loop(0, KP // 2 - 1, _step, 0)
        pltpu.make_async_copy(h_hbm.at[srcv.at[KP - 2]], rows0, sem0).wait()
        pltpu.sync_copy(rows0, agg_s.at[dstv.at[KP - 2]], add=True)
        pltpu.make_async_copy(h_hbm.at[srcv.at[KP - 1]], rows1, sem1).wait()
        pltpu.sync_copy(rows1, agg_s.at[dstv.at[KP - 1]], add=True)
    plsc.subcore_barrier()

    off = 0
    for sz in ROW_CH:
        pltpu.sync_copy(agg_s.at[pl.ds(base + off, sz)], rows0.at[pl.ds(0, sz)])
        pltpu.sync_copy(rows0.at[pl.ds(0, sz)], out_hbm.at[c, pl.ds(base + off, sz)])
        off += sz


def _make_agg_call():
    return pl.kernel(
        _agg_body,
        out_type=jax.ShapeDtypeStruct((NC, P, D), _f32),
        mesh=_mesh(),
        scratch_types=[
            pltpu.VMEM((KP, CHUNK), jnp.int32),
            pltpu.VMEM((KP, CHUNK), jnp.int32),
            pltpu.VMEM((CHUNK, D), _f32),
            pltpu.VMEM((CHUNK, D), _f32),
            pltpu.VMEM_SHARED((P, D), _f32),
            pltpu.SemaphoreType.DMA,
            pltpu.SemaphoreType.DMA,
        ],
    )


def _norm_from(dref):
    deg = dref[0, :, 0:1] + dref[1, :, 0:1]
    return jnp.where(deg > 0, lax.rsqrt(jnp.maximum(deg, 1.0)), 0.0)


def _m1_body(x_ref, w_ref, do_ref, o_ref):
    no = _norm_from(do_ref)
    o_ref[:, :] = jnp.dot(x_ref[:, :], w_ref[:, :],
                          preferred_element_type=_f32) * no


def _m2_body(p_ref, di_ref, b_ref, w_ref, do_ref, o_ref):
    agg = p_ref[0] + p_ref[1]
    h = jnp.maximum(agg * _norm_from(di_ref) + b_ref[:, :], 0.0)
    o_ref[:, :] = jnp.dot(h, w_ref[:, :], preferred_element_type=_f32) \
        * _norm_from(do_ref)


def _fin_body(p_ref, di_ref, b_ref, o_ref):
    agg = p_ref[0] + p_ref[1]
    o_ref[:, :] = jnp.maximum(agg * _norm_from(di_ref) + b_ref[:, :], 0.0)


_spec_rows = pl.BlockSpec((BR, D), lambda i: (i, 0))
_spec_w = pl.BlockSpec((D, D), lambda i: (0, 0))
_spec_parts = pl.BlockSpec((2, BR, D), lambda i: (0, i, 0))
_spec_b = pl.BlockSpec((1, D), lambda i: (0, 0))
_out_rows = jax.ShapeDtypeStruct((N, D), _f32)

_m1_call = pl.pallas_call(
    _m1_body, grid=(N // BR,),
    in_specs=[_spec_rows, _spec_w, _spec_parts],
    out_specs=_spec_rows, out_shape=_out_rows)

_m2_call = pl.pallas_call(
    _m2_body, grid=(N // BR,),
    in_specs=[_spec_parts, _spec_parts, _spec_b, _spec_w, _spec_parts],
    out_specs=_spec_rows, out_shape=_out_rows)

_fin_call = pl.pallas_call(
    _fin_body, grid=(N // BR,),
    in_specs=[_spec_parts, _spec_parts, _spec_b],
    out_specs=_spec_rows, out_shape=_out_rows)


def kernel(x, edge_index, W1, b1, W2, b2):
    ei = edge_index.astype(jnp.int32)
    src, dst = ei[0], ei[1]
    pad = E_PAD - E
    src_g = jnp.concatenate([src, jnp.zeros((pad,), jnp.int32)])
    src_d = jnp.concatenate([src, jnp.full((pad,), DUMMY, jnp.int32)])
    dst_p = jnp.concatenate([dst, jnp.full((pad,), DUMMY, jnp.int32)])
    src_g = src_g.reshape(NW, K, CHUNK)
    src_d = src_d.reshape(NW, K, CHUNK)
    dst_p = dst_p.reshape(NW, K, CHUNK)

    deg_call = _make_deg_call()
    agg_call = _make_agg_call()

    dout_p = deg_call(src_d)
    din_p = deg_call(dst_p)
    dout_c = dout_p[:, :N]
    din_c = din_p[:, :N]

    h1 = _m1_call(x, W1, dout_c)
    parts1 = agg_call(h1, src_g, dst_p)
    h2 = _m2_call(parts1[:, :N], din_c, b1.reshape(1, D), W2, dout_c)
    parts2 = agg_call(h2, src_g, dst_p)
    return _fin_call(parts2[:, :N], din_c, b2.reshape(1, D))
